# baseline (speedup 1.0000x reference)
import functools

import jax
import jax.numpy as jnp
import numpy as np
from jax import lax
from jax.experimental import pallas as pl
from jax.experimental.pallas import tpu as pltpu
from jax.experimental.pallas import tpu_sc as plsc

IC = 512
NGROUPS = 32
CPG = IC // NGROUPS
EPS = 1e-5
NB = 16

_PREC = jax.lax.Precision.HIGHEST


def _pick_rows(n):
    for r in (2000, 1000, 800, 400, 200, 80, 40, 8):
        if n % r == 0:
            return r
    return n


def _onehot_t(ids_ref, rows):
    ids = ids_ref[...].reshape(1, rows)
    biota = jax.lax.broadcasted_iota(jnp.int32, (NB, rows), 0)
    return (ids == biota).astype(jnp.bfloat16)


def _stats_kernel(ids_ref, x_ref, s1_ref, s2_ref, *, rows):
    x = x_ref[...]
    oht = _onehot_t(ids_ref, rows)
    dims = (((1,), (0,)), ((), ()))
    s1 = jax.lax.dot_general(oht, x.astype(jnp.bfloat16), dims,
                             preferred_element_type=jnp.float32)
    s2 = jax.lax.dot_general(oht, (x * x).astype(jnp.bfloat16), dims,
                             preferred_element_type=jnp.float32)

    @pl.when(pl.program_id(0) == 0)
    def _init():
        s1_ref[...] = s1
        s2_ref[...] = s2

    @pl.when(pl.program_id(0) != 0)
    def _acc():
        s1_ref[...] += s1
        s2_ref[...] += s2


_SCC = 800
_NW = 32


def _sc_counts(ids_hbm, cntp_hbm, ids_v, cnt_v, *, nrows):
    wid = lax.axis_index("s") * 2 + lax.axis_index("c")
    nchunks = nrows // _SCC
    nk = (nchunks - wid + _NW - 1) // _NW
    zf = jnp.zeros((16,), jnp.float32)
    onef = jnp.ones((16,), jnp.float32)
    for b in range(NB):
        cnt_v[pl.ds(b * 16, 16)] = zf

    def do_chunk(i, carry):
        r0 = (wid + i * _NW) * _SCC
        pltpu.sync_copy(ids_hbm.at[pl.ds(r0, _SCC)], ids_v.at[pl.ds(0, _SCC)])

        def grp(g, a):
            idvec = ids_v[pl.ds(g * 16, 16)]
            return tuple(
                ab + jnp.where(idvec == b, onef, zf)
                for b, ab in enumerate(a)
            )

        accs = lax.fori_loop(0, _SCC // 16, grp, (zf,) * NB)
        for b in range(NB):
            cnt_v[pl.ds(b * 16, 16)] = cnt_v[pl.ds(b * 16, 16)] + accs[b]
        return carry

    lax.fori_loop(0, nk, do_chunk, 0)
    pltpu.sync_copy(cnt_v, cntp_hbm.at[wid])


def _table_kernel(s1_ref, s2_ref, cntp_ref, g_ref, w_ref, b_ref,
                  scale_ref, shift_ref):
    s1 = s1_ref[...]
    s2 = s2_ref[...]
    cnt = cntp_ref[0:NB, :]
    for t in range(1, _NW):
        cnt = cnt + cntp_ref[pl.ds(NB * t, NB), :]
    n16 = jnp.sum(cnt, axis=1, keepdims=True) * CPG
    ic = 1.0 / (n16 + EPS)
    dims = (((1,), (0,)), ((), ()))
    s1g = jax.lax.dot_general(s1, g_ref[...], dims,
                              preferred_element_type=jnp.float32,
                              precision=_PREC)
    s2g = jax.lax.dot_general(s2, g_ref[...], dims,
                              preferred_element_type=jnp.float32,
                              precision=_PREC)
    m = s1g * ic
    q = n16 * ic
    var = s2g * ic - m * m * (2.0 - q)
    inv_std = jax.lax.rsqrt(var + EPS)
    scale = inv_std * w_ref[...]
    scale_ref[...] = scale
    shift_ref[...] = b_ref[...] - m * scale


def _norm_kernel(ids_ref, x_ref, scale_ref, shift_ref, o_ref, *, rows):
    oht = _onehot_t(ids_ref, rows)
    dims = (((0,), (0,)), ((), ()))

    def expand(tbl):
        hi = tbl.astype(jnp.bfloat16)
        lo = (tbl - hi.astype(jnp.float32)).astype(jnp.bfloat16)
        out = jax.lax.dot_general(oht, hi, dims,
                                  preferred_element_type=jnp.float32)
        return out + jax.lax.dot_general(oht, lo, dims,
                                         preferred_element_type=jnp.float32)

    sc = expand(scale_ref[...])
    sh = expand(shift_ref[...])
    o_ref[...] = x_ref[...] * sc + sh


def kernel(data, batch_id, weights, bias):
    n = data.shape[0]
    rows = _pick_rows(n)
    nblk = n // rows
    ids = batch_id.astype(jnp.int32).reshape(nblk, 1, rows)

    ids_flat = batch_id.astype(jnp.int32)
    mesh = plsc.VectorSubcoreMesh(core_axis_name="c", subcore_axis_name="s")
    sc_counts = functools.partial(
        pl.kernel,
        mesh=mesh,
        out_type=jax.ShapeDtypeStruct((_NW, NB * NB), jnp.float32),
        scratch_types=[
            pltpu.VMEM((_SCC,), jnp.int32),
            pltpu.VMEM((NB * NB,), jnp.float32),
        ],
    )(functools.partial(_sc_counts, nrows=n))
    cntp = sc_counts(ids_flat).reshape(_NW * NB, NB)

    s1, s2 = pl.pallas_call(
        functools.partial(_stats_kernel, rows=rows),
        grid=(nblk,),
        in_specs=[
            pl.BlockSpec((1, 1, rows), lambda i: (i, 0, 0)),
            pl.BlockSpec((rows, IC), lambda i: (i, 0)),
        ],
        out_specs=[
            pl.BlockSpec((NB, IC), lambda i: (0, 0)),
            pl.BlockSpec((NB, IC), lambda i: (0, 0)),
        ],
        out_shape=[
            jax.ShapeDtypeStruct((NB, IC), jnp.float32),
            jax.ShapeDtypeStruct((NB, IC), jnp.float32),
        ],
    )(ids, data)

    gmat = jnp.asarray(np.kron(np.eye(NGROUPS, dtype=np.float32),
                               np.ones((CPG, CPG), np.float32)))
    scale, shift = pl.pallas_call(
        _table_kernel,
        out_shape=[
            jax.ShapeDtypeStruct((NB, IC), jnp.float32),
            jax.ShapeDtypeStruct((NB, IC), jnp.float32),
        ],
    )(s1, s2, cntp, gmat, weights, bias)

    out = pl.pallas_call(
        functools.partial(_norm_kernel, rows=rows),
        grid=(nblk,),
        in_specs=[
            pl.BlockSpec((1, 1, rows), lambda i: (i, 0, 0)),
            pl.BlockSpec((rows, IC), lambda i: (i, 0)),
            pl.BlockSpec((NB, IC), lambda i: (0, 0)),
            pl.BlockSpec((NB, IC), lambda i: (0, 0)),
        ],
        out_specs=pl.BlockSpec((rows, IC), lambda i: (i, 0)),
        out_shape=jax.ShapeDtypeStruct((n, IC), jnp.float32),
    )(ids, data, scale, shift)
    return out


# speedup vs baseline: 1.1372x; 1.1372x over previous
import functools

import jax
import jax.numpy as jnp
import numpy as np
from jax import lax
from jax.experimental import pallas as pl
from jax.experimental.pallas import tpu as pltpu
from jax.experimental.pallas import tpu_sc as plsc

IC = 512
NGROUPS = 32
CPG = IC // NGROUPS
EPS = 1e-5
NB = 16

_PREC = jax.lax.Precision.HIGHEST


def _pick_rows(n):
    for r in (5000, 4000, 2000, 1000, 800, 400, 200, 80, 40, 8):
        if n % r == 0:
            return r
    return n


def _onehot_t(ids_ref, rows):
    ids = ids_ref[...].reshape(1, rows)
    biota = jax.lax.broadcasted_iota(jnp.int32, (NB, rows), 0)
    return (ids == biota).astype(jnp.bfloat16)


def _stats_kernel(ids_ref, x_ref, s1_ref, s2_ref, *, rows):
    x = x_ref[...]
    oht = _onehot_t(ids_ref, rows)
    dims = (((1,), (0,)), ((), ()))
    s1 = jax.lax.dot_general(oht, x.astype(jnp.bfloat16), dims,
                             preferred_element_type=jnp.float32)
    s2 = jax.lax.dot_general(oht, (x * x).astype(jnp.bfloat16), dims,
                             preferred_element_type=jnp.float32)

    @pl.when(pl.program_id(0) == 0)
    def _init():
        s1_ref[...] = s1
        s2_ref[...] = s2

    @pl.when(pl.program_id(0) != 0)
    def _acc():
        s1_ref[...] += s1
        s2_ref[...] += s2


_SCC = 800
_NW = 32


def _sc_counts(ids_hbm, cntp_hbm, ids_v, cnt_v, *, nrows):
    wid = lax.axis_index("s") * 2 + lax.axis_index("c")
    nchunks = nrows // _SCC
    nk = (nchunks - wid + _NW - 1) // _NW
    zf = jnp.zeros((16,), jnp.float32)
    onef = jnp.ones((16,), jnp.float32)
    for b in range(NB):
        cnt_v[pl.ds(b * 16, 16)] = zf

    def do_chunk(i, carry):
        r0 = (wid + i * _NW) * _SCC
        pltpu.sync_copy(ids_hbm.at[pl.ds(r0, _SCC)], ids_v.at[pl.ds(0, _SCC)])

        def grp(g, a):
            idvec = ids_v[pl.ds(g * 16, 16)]
            return tuple(
                ab + jnp.where(idvec == b, onef, zf)
                for b, ab in enumerate(a)
            )

        accs = lax.fori_loop(0, _SCC // 16, grp, (zf,) * NB)
        for b in range(NB):
            cnt_v[pl.ds(b * 16, 16)] = cnt_v[pl.ds(b * 16, 16)] + accs[b]
        return carry

    lax.fori_loop(0, nk, do_chunk, 0)
    pltpu.sync_copy(cnt_v, cntp_hbm.at[wid])


def _table_kernel(s1_ref, s2_ref, cntp_ref, g_ref, w_ref, b_ref,
                  scale_ref, shift_ref):
    s1 = s1_ref[...]
    s2 = s2_ref[...]
    cnt = cntp_ref[0:NB, :]
    for t in range(1, _NW):
        cnt = cnt + cntp_ref[pl.ds(NB * t, NB), :]
    n16 = jnp.sum(cnt, axis=1, keepdims=True) * CPG
    ic = 1.0 / (n16 + EPS)
    dims = (((1,), (0,)), ((), ()))
    s1g = jax.lax.dot_general(s1, g_ref[...], dims,
                              preferred_element_type=jnp.float32,
                              precision=_PREC)
    s2g = jax.lax.dot_general(s2, g_ref[...], dims,
                              preferred_element_type=jnp.float32,
                              precision=_PREC)
    m = s1g * ic
    q = n16 * ic
    var = s2g * ic - m * m * (2.0 - q)
    inv_std = jax.lax.rsqrt(var + EPS)
    scale = inv_std * w_ref[...]
    scale_ref[...] = scale
    shift_ref[...] = b_ref[...] - m * scale


def _norm_kernel(ids_ref, x_ref, scale_ref, shift_ref, o_ref, *, rows):
    oht = _onehot_t(ids_ref, rows)
    dims = (((0,), (0,)), ((), ()))

    def expand(tbl):
        hi = tbl.astype(jnp.bfloat16)
        lo = (tbl - hi.astype(jnp.float32)).astype(jnp.bfloat16)
        out = jax.lax.dot_general(oht, hi, dims,
                                  preferred_element_type=jnp.float32)
        return out + jax.lax.dot_general(oht, lo, dims,
                                         preferred_element_type=jnp.float32)

    sc = expand(scale_ref[...])
    sh = expand(shift_ref[...])
    o_ref[...] = x_ref[...] * sc + sh


def kernel(data, batch_id, weights, bias):
    n = data.shape[0]
    rows = _pick_rows(n)
    nblk = n // rows
    ids = batch_id.astype(jnp.int32).reshape(nblk, 1, rows)

    ids_flat = batch_id.astype(jnp.int32)
    mesh = plsc.VectorSubcoreMesh(core_axis_name="c", subcore_axis_name="s")
    sc_counts = functools.partial(
        pl.kernel,
        mesh=mesh,
        out_type=jax.ShapeDtypeStruct((_NW, NB * NB), jnp.float32),
        scratch_types=[
            pltpu.VMEM((_SCC,), jnp.int32),
            pltpu.VMEM((NB * NB,), jnp.float32),
        ],
    )(functools.partial(_sc_counts, nrows=n))
    cntp = sc_counts(ids_flat).reshape(_NW * NB, NB)

    s1, s2 = pl.pallas_call(
        functools.partial(_stats_kernel, rows=rows),
        grid=(nblk,),
        in_specs=[
            pl.BlockSpec((1, 1, rows), lambda i: (i, 0, 0)),
            pl.BlockSpec((rows, IC), lambda i: (i, 0)),
        ],
        out_specs=[
            pl.BlockSpec((NB, IC), lambda i: (0, 0)),
            pl.BlockSpec((NB, IC), lambda i: (0, 0)),
        ],
        out_shape=[
            jax.ShapeDtypeStruct((NB, IC), jnp.float32),
            jax.ShapeDtypeStruct((NB, IC), jnp.float32),
        ],
    )(ids, data)

    gmat = jnp.asarray(np.kron(np.eye(NGROUPS, dtype=np.float32),
                               np.ones((CPG, CPG), np.float32)))
    scale, shift = pl.pallas_call(
        _table_kernel,
        out_shape=[
            jax.ShapeDtypeStruct((NB, IC), jnp.float32),
            jax.ShapeDtypeStruct((NB, IC), jnp.float32),
        ],
    )(s1, s2, cntp, gmat, weights, bias)

    out = pl.pallas_call(
        functools.partial(_norm_kernel, rows=rows),
        grid=(nblk,),
        in_specs=[
            pl.BlockSpec((1, 1, rows), lambda i: (i, 0, 0)),
            pl.BlockSpec((rows, IC), lambda i: (i, 0)),
            pl.BlockSpec((NB, IC), lambda i: (0, 0)),
            pl.BlockSpec((NB, IC), lambda i: (0, 0)),
        ],
        out_specs=pl.BlockSpec((rows, IC), lambda i: (i, 0)),
        out_shape=jax.ShapeDtypeStruct((n, IC), jnp.float32),
    )(ids, data, scale, shift)
    return out
